# Initial kernel scaffold; baseline (speedup 1.0000x reference)
import jax
import jax.numpy as jnp
from jax.experimental import pallas as pl


def kernel(xyz, features, point_pose, point_pose_mask, W0, b0, W1, b1, W2, b2):
    raise NotImplementedError("write your pallas kernel here")


# speedup vs baseline: 10.5727x; 10.5727x over previous
import functools

import jax
import jax.numpy as jnp
import numpy as np
from jax import lax
from jax.experimental import pallas as pl
from jax.experimental.pallas import tpu as pltpu
from jax.experimental.pallas import tpu_sc as plsc

_B = 4; _N = 8192; _P = 1024; _S = 64; _R = 0.2; _CIN = 16
_BN_EPS = 1e-5
_NW = 32
_CPW = (_B * _P) // _NW
_W = _N // 16
_D = 32
_IW = 80


def _fps_body(x_ref, inds_ref, nx_ref, mind_ref):
    mind_ref[...] = jnp.full((_B, 64, 128), 1e10, jnp.float32)
    iota2 = (lax.broadcasted_iota(jnp.int32, (64, 128), 0) * 128
             + lax.broadcasted_iota(jnp.int32, (64, 128), 1))
    lane = lax.broadcasted_iota(jnp.int32, (1, 128), 1)

    zrow = jnp.zeros((1, 128), jnp.int32)
    inds_ref[0:1, :] = zrow
    oh0 = iota2 == 0
    init = []
    for b in range(_B):
        cs = tuple(jnp.sum(jnp.where(oh0, x_ref[b, d], 0.0)) for d in range(3))
        init.append(cs)
    init = tuple(init)
    for d in range(3):
        coord0 = jnp.zeros((1, 128), jnp.float32)
        for b in range(_B):
            coord0 = jnp.where(lane == b, init[b][d], coord0)
        nx_ref[d, 0:1, :] = coord0

    def body(i, carry):
        coords = carry
        nxts = []
        for b in range(_B):
            cx, cy, cz = coords[b]
            xb = x_ref[b, 0]
            yb = x_ref[b, 1]
            zb = x_ref[b, 2]
            dx = xb - cx
            dy = yb - cy
            dz = zb - cz
            d = (dx * dx + dy * dy) + dz * dz
            m = jnp.minimum(mind_ref[b], d)
            mind_ref[b] = m
            mx = jnp.max(m)
            nxt = jnp.min(jnp.where(m == mx, iota2, _N))
            nxts.append(nxt)
        newcoords = []
        for b in range(_B):
            oh = iota2 == nxts[b]
            cx = jnp.sum(jnp.where(oh, x_ref[b, 0], 0.0))
            cy = jnp.sum(jnp.where(oh, x_ref[b, 1], 0.0))
            cz = jnp.sum(jnp.where(oh, x_ref[b, 2], 0.0))
            newcoords.append((cx, cy, cz))
        row = jnp.zeros((1, 128), jnp.int32)
        for b in range(_B):
            row = jnp.where(lane == b, nxts[b], row)
        inds_ref[pl.ds(i, 1), :] = row
        for d in range(3):
            crow = jnp.zeros((1, 128), jnp.float32)
            for b in range(_B):
                crow = jnp.where(lane == b, newcoords[b][d], crow)
            nx_ref[d, pl.ds(i, 1), :] = crow
        return tuple(newcoords)

    lax.fori_loop(1, _P, body, init)


def _run_fps(xyz):
    xq = jnp.transpose(xyz, (0, 2, 1)).reshape(_B, 3, 64, 128)
    inds_pk, nx_pk = pl.pallas_call(
        _fps_body,
        out_shape=[jax.ShapeDtypeStruct((_P, 128), jnp.int32),
                   jax.ShapeDtypeStruct((3, _P, 128), jnp.float32)],
        scratch_shapes=[pltpu.VMEM((_B, 64, 128), jnp.float32)],
    )(xq)
    inds = jnp.transpose(inds_pk[:, :_B], (1, 0))
    new_xyz = jnp.transpose(nx_pk[:, :, :_B], (2, 1, 0))
    return inds, new_xyz


_MCB = 128
_NB = 512


def _mask_body(c_ref, xt_ref, pk_ref, pc_ref, tri_ref, o_ref):
    cb = c_ref[0]
    xb = xt_ref[0]
    e = jnp.dot(cb, xb, preferred_element_type=jnp.float32)
    c0 = cb[:, 0:1]; c1 = cb[:, 1:2]; c2 = cb[:, 2:3]
    cs = (c0 * c0 + c1 * c1) + c2 * c2
    x0 = xb[0:1, :]; x1 = xb[1:2, :]; x2 = xb[2:3, :]
    xs = (x0 * x0 + x1 * x1) + x2 * x2
    d2 = (cs + xs) - 2.0 * e
    valid = (d2 <= jnp.float32(_R * _R)).astype(jnp.float32)
    pk = pk_ref[...]
    pc = pc_ref[...]
    bits, counts = [], []
    for nc in range(_N // _NB):
        v = valid[:, nc * _NB:(nc + 1) * _NB]
        bits.append(jnp.dot(v, pk, preferred_element_type=jnp.float32))
        counts.append(jnp.dot(v, pc, preferred_element_type=jnp.float32))
    bits = jnp.concatenate(bits, axis=1)
    counts = jnp.concatenate(counts, axis=1)
    csum = jnp.dot(counts, tri_ref[...],
                   preferred_element_type=jnp.float32)
    excl = (csum - counts).astype(jnp.int32)
    o_ref[0] = bits.astype(jnp.int32) + (excl << 16)


def _run_mask(new_xyz, xyzT):
    i = np.arange(_NB)
    pk = np.zeros((_NB, _NB // 16), np.float32)
    pk[i, i // 16] = np.float32(2.0) ** (i % 16)
    pc = np.zeros((_NB, _NB // 16), np.float32)
    pc[i, i // 16] = 1.0
    tri = np.triu(np.ones((_W, _W), np.float32))
    return pl.pallas_call(
        _mask_body,
        grid=(_B, _P // _MCB),
        in_specs=[
            pl.BlockSpec((1, _MCB, 3), lambda b, p: (b, p, 0)),
            pl.BlockSpec((1, 3, _N), lambda b, p: (b, 0, 0)),
            pl.BlockSpec((_NB, _NB // 16), lambda b, p: (0, 0)),
            pl.BlockSpec((_NB, _NB // 16), lambda b, p: (0, 0)),
            pl.BlockSpec((_W, _W), lambda b, p: (0, 0)),
        ],
        out_specs=pl.BlockSpec((1, _MCB, _W), lambda b, p: (b, p, 0)),
        out_shape=jax.ShapeDtypeStruct((_B, _P, _W), jnp.int32),
    )(new_xyz, xyzT, jnp.asarray(pk), jnp.asarray(pc), jnp.asarray(tri))


def _make_luts():
    lutpc = np.zeros((256,), np.int32)
    lutsel = np.zeros((2048,), np.int32)
    for b in range(256):
        bitpos = [p for p in range(8) if (b >> p) & 1]
        lutpc[b] = len(bitpos)
        for r, p in enumerate(bitpos):
            lutsel[b * 8 + r] = p
    return jnp.asarray(lutpc), jnp.asarray(lutsel)


def _make_lutcc():
    lut = np.zeros((4096,), np.int32)
    for b in range(256):
        pos = [p for p in range(8) if (b >> p) & 1]
        for r, p in enumerate(pos):
            lut[b * 16 + r] = p
        lut[b * 16 + 8] = len(pos) - 8
    return jnp.asarray(lut)


def _sc_compact_gather(packed, table, lutcc):
    mesh = plsc.VectorSubcoreMesh(core_axis_name="c", subcore_axis_name="s")

    @functools.partial(
        pl.kernel,
        out_type=[jax.ShapeDtypeStruct((_B * _P, _S, _D), jnp.float32),
                  jax.ShapeDtypeStruct((_B * _P * 16,), jnp.int32)],
        mesh=mesh,
        compiler_params=pltpu.CompilerParams(use_tc_tiling_on_sc=False),
        scratch_types=[
            pltpu.VMEM((_CPW, _W), jnp.int32),
            pltpu.VMEM((_CPW * _S + 32,), jnp.int32),
            pltpu.VMEM((4096,), jnp.int32),
            pltpu.VMEM((_CPW * 16,), jnp.int32),
            pltpu.VMEM((_S, _D), jnp.float32),
            pltpu.SemaphoreType.DMA,
        ],
    )
    def k(pk_hbm, table_hbm, lut_hbm, out_hbm, cnt_hbm,
          bm_v, idx_v, lut_v, cnts_v, gbuf, sem):
        wid = lax.axis_index("s") * 2 + lax.axis_index("c")
        base_c = wid * _CPW
        base_row = (base_c // _P) * _N
        pltpu.sync_copy(pk_hbm.at[pl.ds(base_c, _CPW)], bm_v)
        pltpu.sync_copy(lut_hbm, lut_v)

        lanes = lax.iota(jnp.int32, 16)

        @pl.loop(0, (_CPW * _S + 32) // 16)
        def init(i):
            idx_v[pl.ds(i * 16, 16)] = lanes + base_row

        @pl.loop(0, _CPW)
        def per_center(ci):
            @pl.loop(0, _W // 16, init_carry=jnp.int32(0))
            def cnt(g, cnt):
                wv16 = bm_v[ci, pl.ds(g * 16, 16)]
                for t in range(16):
                    w0 = wv16[t]
                    lo = w0 & 0xFF
                    hi = lax.shift_right_logical(w0, 8) & 0xFF
                    rlo = lut_v[pl.ds(lo * 16, 16)]
                    rhi = lut_v[pl.ds(hi * 16, 16)]
                    klo = rlo[8] + 8
                    khi = rhi[8] + 8
                    base_w = (g * 16 + t) * 16 + base_row
                    off0 = ci * _S + jnp.minimum(cnt, jnp.int32(_S))
                    idx_v[pl.ds(off0, 16)] = rlo + base_w
                    idx_v[pl.ds(off0 + klo, 16)] = rhi + (base_w + 8)
                    cnt = cnt + klo + khi
                return cnt

            cnts_v[pl.ds(ci * 16, 16)] = cnt + lanes
            pltpu.async_copy(table_hbm.at[idx_v.at[pl.ds(ci * _S, _S)]],
                             gbuf, sem).wait()
            pltpu.sync_copy(gbuf, out_hbm.at[base_c + ci])

        pltpu.sync_copy(cnts_v, cnt_hbm.at[pl.ds(base_c * 16, _CPW * 16)])

    return k(packed, table, lutcc)


_CB = 128


def _mlp_body(g_ref, cnt_ref, cent_ref, w0_ref, b0_ref, w1_ref, b1_ref,
              w2_ref, b2_ref, nf_ref, mp_ref):
    g3 = g_ref[...]
    kk = cnt_ref[...][:, 0][:, None, None]
    sid = lax.broadcasted_iota(jnp.int32, (_CB, _S, 1), 1)
    g3 = jnp.where(sid < kk, g3, jnp.broadcast_to(g3[:, 0:1, :], (_CB, _S, _D)))
    flat = g3.reshape(_CB * _S, _D)
    pose = flat[:, 3:4]
    mask = flat[:, 4:5]
    sp = (mask * jnp.sin(pose)).reshape(_CB, _S)
    cp = (mask * jnp.cos(pose)).reshape(_CB, _S)
    s = jnp.sum(sp, axis=1, keepdims=True)
    c = jnp.sum(cp, axis=1, keepdims=True)
    cnt = jnp.sum(mask.reshape(_CB, _S), axis=1, keepdims=True)
    has = cnt > 0
    mode = jnp.arctan2(jnp.where(has, s, 0.0), jnp.where(has, c, 1.0))
    mp_ref[...] = mode
    ct = jnp.broadcast_to(jnp.cos(mode)[:, :, None], (_CB, _S, 1)).reshape(_CB * _S, 1)
    st = jnp.broadcast_to(jnp.sin(mode)[:, :, None], (_CB, _S, 1)).reshape(_CB * _S, 1)

    centc = jnp.broadcast_to(cent_ref[...][:, None, :], (_CB, _S, 3))
    rel = (flat[:, 0:3] - centc.reshape(_CB * _S, 3)) / jnp.float32(_R)
    r0 = rel[:, 0:1]; r1 = rel[:, 1:2]; r2 = rel[:, 2:3]
    xr = ct * r0 + st * r1
    yr = -st * r0 + ct * r1
    feats = flat[:, 5:5 + _CIN]
    x = jnp.concatenate([xr, yr, r2, feats], axis=1)

    scale = jnp.float32(1.0) / jnp.sqrt(jnp.float32(1.0 + _BN_EPS))

    def layer(h, w_ref, b_ref):
        y = jnp.dot(h, w_ref[...], preferred_element_type=jnp.float32)
        y = (y + b_ref[...]) * scale
        return jnp.maximum(y, 0.0)

    h = layer(x, w0_ref, b0_ref)
    h = layer(h, w1_ref, b1_ref)
    h = layer(h, w2_ref, b2_ref)
    nf_ref[...] = jnp.max(h.reshape(_CB, _S, 64), axis=1)


def _run_mlp(gathered, cnts, cent, w0t, b0, w1t, b1, w2t, b2):
    nc = (_B * _P) // _CB
    return pl.pallas_call(
        _mlp_body,
        grid=(nc,),
        in_specs=[
            pl.BlockSpec((_CB, _S, _D), lambda i: (i, 0, 0)),
            pl.BlockSpec((_CB, 16), lambda i: (i, 0)),
            pl.BlockSpec((_CB, 3), lambda i: (i, 0)),
            pl.BlockSpec((19, 32), lambda i: (0, 0)),
            pl.BlockSpec((1, 32), lambda i: (0, 0)),
            pl.BlockSpec((32, 32), lambda i: (0, 0)),
            pl.BlockSpec((1, 32), lambda i: (0, 0)),
            pl.BlockSpec((32, 64), lambda i: (0, 0)),
            pl.BlockSpec((1, 64), lambda i: (0, 0)),
        ],
        out_specs=[
            pl.BlockSpec((_CB, 64), lambda i: (i, 0)),
            pl.BlockSpec((_CB, 1), lambda i: (i, 0)),
        ],
        out_shape=[jax.ShapeDtypeStruct((_B * _P, 64), jnp.float32),
                   jax.ShapeDtypeStruct((_B * _P, 1), jnp.float32)],
    )(gathered, cnts, cent, w0t, b0, w1t, b1, w2t, b2)


def kernel(xyz, features, point_pose, point_pose_mask, W0, b0, W1, b1, W2, b2):
    inds, new_xyz = _run_fps(xyz)

    xyzT = jnp.transpose(xyz, (0, 2, 1))
    packed = _run_mask(new_xyz, xyzT)

    table = jnp.concatenate(
        [xyz,
         point_pose[..., None],
         point_pose_mask[..., None],
         jnp.transpose(features, (0, 2, 1)),
         jnp.zeros((_B, _N, _D - 5 - _CIN), jnp.float32)], axis=-1)
    table = table.reshape(_B * _N, _D)

    gathered, cnts = _sc_compact_gather(packed.reshape(_B * _P, _W), table,
                                        _make_lutcc())
    cnts = cnts.reshape(_B * _P, 16)

    nf, mp = _run_mlp(gathered, cnts, new_xyz.reshape(_B * _P, 3),
                      jnp.transpose(W0, (1, 0)), b0[None, :],
                      jnp.transpose(W1, (1, 0)), b1[None, :],
                      jnp.transpose(W2, (1, 0)), b2[None, :])

    new_features = jnp.transpose(nf.reshape(_B, _P, 64), (0, 2, 1))
    mode_pose = mp.reshape(_B, _P)
    return (new_xyz, new_features, inds.astype(jnp.int64), mode_pose)
